# Initial kernel scaffold; baseline (speedup 1.0000x reference)
import jax
import jax.numpy as jnp
from jax.experimental import pallas as pl


def kernel(rs, cs, u_outs, u_ins, time_steps, r_table, c_table, u_out_table, W1, b1, W2, b2, ln_g, ln_b, t2v_w0, t2v_b0, t2v_W, t2v_B):
    raise NotImplementedError("write your pallas kernel here")


# speedup vs baseline: 1.9776x; 1.9776x over previous
import jax
import jax.numpy as jnp
from jax.experimental import pallas as pl
from jax.experimental.pallas import tpu as pltpu

_H = 128
_RB = 1024


def _body(rs_ref, cs_ref, uo_ref, u_ref, t_ref,
          rt_ref, ct_ref, ut_ref, w1_ref, b1_ref, w2_ref, b2_ref,
          g_ref, be_ref, t2vw_ref, t2vb_ref,
          r_out, c_out, uo_out, uin_out, t2v_out):
    def sel3(idx, tab_ref):
        t0 = tab_ref[0:1, :]
        t1 = tab_ref[1:2, :]
        t2 = tab_ref[2:3, :]
        return jnp.where(idx == 0, t0, jnp.where(idx == 1, t1, t2))

    rs = rs_ref[...]
    cs = cs_ref[...]
    uo = uo_ref[...]
    r_out[...] = sel3(rs, rt_ref)
    c_out[...] = sel3(cs, ct_ref)
    u0 = ut_ref[0:1, :]
    u1 = ut_ref[1:2, :]
    uo_out[...] = jnp.where(uo == 0, u0, u1)

    u = u_ref[...]
    h = u * w1_ref[...] + b1_ref[...]
    h = h * jax.nn.sigmoid(h)
    o = jnp.dot(h, w2_ref[...], preferred_element_type=jnp.float32)
    o = o + b2_ref[...]
    mu = jnp.mean(o, axis=-1, keepdims=True)
    d = o - mu
    var = jnp.mean(d * d, axis=-1, keepdims=True)
    uin_out[...] = d * jax.lax.rsqrt(var + 1e-5) * g_ref[...] + be_ref[...]

    t = t_ref[...]
    z = t * t2vw_ref[...] + t2vb_ref[...]
    lane = jax.lax.broadcasted_iota(jnp.int32, z.shape, 1)
    t2v_out[...] = jnp.where(lane == 0, z, jnp.sin(z))


def kernel(rs, cs, u_outs, u_ins, time_steps, r_table, c_table, u_out_table,
           W1, b1, W2, b2, ln_g, ln_b, t2v_w0, t2v_b0, t2v_W, t2v_B):
    B, L = rs.shape
    N = B * L
    H = r_table.shape[1]
    nb = N // _RB

    col = lambda x: x.reshape(N, 1)
    rs2 = col(rs.astype(jnp.int32))
    cs2 = col(cs.astype(jnp.int32))
    uo2 = col(u_outs.astype(jnp.int32))
    u2 = col(u_ins)
    t2 = col(time_steps)

    t2v_w = jnp.concatenate([t2v_w0, t2v_W], axis=1)
    t2v_b = jnp.concatenate([t2v_b0, t2v_B], axis=0).reshape(1, H)
    b1r = b1.reshape(1, -1)
    b2r = b2.reshape(1, H)
    gr = ln_g.reshape(1, H)
    ber = ln_b.reshape(1, H)

    row_spec = pl.BlockSpec((_RB, 1), lambda i: (i, 0))
    full = lambda a: pl.BlockSpec(a.shape, lambda i: (0,) * a.ndim)
    out_spec = pl.BlockSpec((_RB, H), lambda i: (i, 0))

    outs = pl.pallas_call(
        _body,
        grid=(nb,),
        in_specs=[row_spec, row_spec, row_spec, row_spec, row_spec,
                  full(r_table), full(c_table), full(u_out_table),
                  full(W1), full(b1r), full(W2), full(b2r),
                  full(gr), full(ber), full(t2v_w), full(t2v_b)],
        out_specs=[out_spec] * 5,
        out_shape=[jax.ShapeDtypeStruct((N, H), jnp.float32)] * 5,
        compiler_params=pltpu.CompilerParams(
            dimension_semantics=("arbitrary",)),
    )(rs2, cs2, uo2, u2, t2, r_table, c_table, u_out_table,
      W1, b1r, W2, b2r, gr, ber, t2v_w, t2v_b)

    return tuple(o.reshape(B, L, H) for o in outs)


# speedup vs baseline: 3.3515x; 1.6948x over previous
import jax
import jax.numpy as jnp
from jax.experimental import pallas as pl
from jax.experimental.pallas import tpu as pltpu

_LP = 56
_BB = 16
_RP = _BB * _LP


def _bcast(row, mat):
    return jax.lax.dot_general(row, mat, (((0,), (0,)), ((), ())),
                               preferred_element_type=jnp.float32)


def _body(rs_ref, cs_ref, uo_ref, u_ref, t_ref,
          rt_ref, ct_ref, ut_ref, w1_ref, b1_ref, w2_ref, b2_ref,
          g_ref, be_ref, t2vw_ref, t2vb_ref,
          r_out, c_out, uo_out, uin_out, t2v_out):
    ones = jnp.ones((1, 128), dtype=jnp.float32)

    def sel3(idx_row, tab_ref):
        f = _bcast(idx_row, ones)
        t0 = tab_ref[0:1, :]
        t1 = tab_ref[1:2, :]
        t2 = tab_ref[2:3, :]
        return jnp.where(f == 0.0, t0, jnp.where(f == 1.0, t1, t2))

    def store(ref, val):
        v = val.reshape(_BB, _LP, 128)
        ref[...] = v[:, :50, :]

    store(r_out, sel3(rs_ref[0], rt_ref))
    store(c_out, sel3(cs_ref[0], ct_ref))
    fo = _bcast(uo_ref[0], ones)
    store(uo_out, jnp.where(fo == 0.0, ut_ref[0:1, :], ut_ref[1:2, :]))

    h = _bcast(u_ref[0], w1_ref[...]) + b1_ref[...]
    h = h * jax.nn.sigmoid(h)
    o = jnp.dot(h, w2_ref[...], preferred_element_type=jnp.float32)
    o = o + b2_ref[...]
    mu = jnp.mean(o, axis=-1, keepdims=True)
    d = o - mu
    var = jnp.mean(d * d, axis=-1, keepdims=True)
    store(uin_out, d * jax.lax.rsqrt(var + 1e-5) * g_ref[...] + be_ref[...])

    z = _bcast(t_ref[0], t2vw_ref[...]) + t2vb_ref[...]
    lane = jax.lax.broadcasted_iota(jnp.int32, z.shape, 1)
    store(t2v_out, jnp.where(lane == 0, z, jnp.sin(z)))


def kernel(rs, cs, u_outs, u_ins, time_steps, r_table, c_table, u_out_table,
           W1, b1, W2, b2, ln_g, ln_b, t2v_w0, t2v_b0, t2v_W, t2v_B):
    B, L = rs.shape
    H = r_table.shape[1]
    nb = B // _BB

    def rows(x):
        xp = jnp.pad(x.astype(jnp.float32), ((0, 0), (0, _LP - L)))
        return xp.reshape(nb, 1, _RP)

    rs2, cs2, uo2, u2, t2 = map(rows, (rs, cs, u_outs, u_ins, time_steps))

    t2v_w = jnp.concatenate([t2v_w0, t2v_W], axis=1)
    t2v_b = jnp.concatenate([t2v_b0, t2v_B], axis=0).reshape(1, H)
    b1r = b1.reshape(1, -1)
    b2r = b2.reshape(1, H)
    gr = ln_g.reshape(1, H)
    ber = ln_b.reshape(1, H)

    row_spec = pl.BlockSpec((1, 1, _RP), lambda i: (i, 0, 0))
    full = lambda a: pl.BlockSpec(a.shape, lambda i: (0,) * a.ndim)
    out_spec = pl.BlockSpec((_BB, L, H), lambda i: (i, 0, 0))

    outs = pl.pallas_call(
        _body,
        grid=(nb,),
        in_specs=[row_spec, row_spec, row_spec, row_spec, row_spec,
                  full(r_table), full(c_table), full(u_out_table),
                  full(W1), full(b1r), full(W2), full(b2r),
                  full(gr), full(ber), full(t2v_w), full(t2v_b)],
        out_specs=[out_spec] * 5,
        out_shape=[jax.ShapeDtypeStruct((B, L, H), jnp.float32)] * 5,
        compiler_params=pltpu.CompilerParams(
            dimension_semantics=("arbitrary",)),
    )(rs2, cs2, uo2, u2, t2, r_table, c_table, u_out_table,
      W1, b1r, W2, b2r, gr, ber, t2v_w, t2v_b)

    return tuple(outs)
